# baseline (speedup 1.0000x reference)
import functools

import jax
import jax.numpy as jnp
from jax import lax
from jax.experimental import pallas as pl
from jax.experimental.pallas import tpu as pltpu
from jax.experimental.pallas import tpu_sc as plsc

_N_FEATURES = 26
_N_VALUES = 38462
_EMBED = 16
_BATCH = 16384
_FLAT = _BATCH * _N_FEATURES

_LANES = 16
_G = 128
_CHUNK = 1664
_NW = 32
_PER_W = _FLAT // _NW
_NCHUNK = _PER_W // _CHUNK
_G_PER_CHUNK = _CHUNK // _G


@functools.cache
def _build(flat, per_w, chunk, nchunk, g_per_chunk):
    mesh = plsc.VectorSubcoreMesh(core_axis_name="c", subcore_axis_name="s")

    @functools.partial(
        pl.kernel,
        mesh=mesh,
        compiler_params=pltpu.CompilerParams(use_tc_tiling_on_sc=False),
        out_type=jax.ShapeDtypeStruct((flat, _EMBED), jnp.float32),
        scratch_types=[
            pltpu.VMEM((per_w,), jnp.int32),
            pltpu.VMEM((chunk, _EMBED), jnp.float32),
            pltpu.SemaphoreType.DMA,
        ],
    )
    def run(x_hbm, table_hbm, out_hbm, idx_v, rows_v, sem):
        wid = lax.axis_index("s") * 2 + lax.axis_index("c")
        wstart = wid * per_w

        pltpu.sync_copy(x_hbm.at[pl.ds(wstart, per_w)], idx_v)

        lane = lax.iota(jnp.int32, _LANES)

        def add_off(j, _):
            s = j * _LANES
            pos = lane + (wstart + s)
            off = lax.rem(pos, _N_FEATURES) * _N_VALUES
            idx_v[pl.ds(s, _LANES)] = idx_v[pl.ds(s, _LANES)] + off
            return _

        lax.fori_loop(0, per_w // _LANES, add_off, None)

        def do_chunk(c, _):
            base = c * chunk
            copies = [
                pltpu.async_copy(
                    table_hbm.at[idx_v.at[pl.ds(base + g * _G, _G)]],
                    rows_v.at[pl.ds(g * _G, _G)],
                    sem,
                )
                for g in range(g_per_chunk)
            ]
            for cp in copies:
                cp.wait()
            pltpu.sync_copy(rows_v, out_hbm.at[pl.ds(wstart + base, chunk)])
            return _

        lax.fori_loop(0, nchunk, do_chunk, None)

    return run


def kernel(x, table):
    xf = x.reshape(-1)
    out = _build(_FLAT, _PER_W, _CHUNK, _NCHUNK, _G_PER_CHUNK)(xf, table)
    return out.reshape(_BATCH, _N_FEATURES, _EMBED)


# speedup vs baseline: 1.2957x; 1.2957x over previous
import functools

import jax
import jax.numpy as jnp
from jax import lax
from jax.experimental import pallas as pl
from jax.experimental.pallas import tpu as pltpu
from jax.experimental.pallas import tpu_sc as plsc

_N_FEATURES = 26
_N_VALUES = 38462
_EMBED = 16
_BATCH = 16384
_FLAT = _BATCH * _N_FEATURES

_LANES = 16
_G = 104
_GB = 4
_CHUNK_B = 64
_NW = 32
_B_PER_W = _BATCH // _NW
_PER_W = _FLAT // _NW
_NCHUNK = _B_PER_W // _CHUNK_B
_G_PER_CHUNK = _CHUNK_B // _GB


@functools.cache
def _build():
    mesh = plsc.VectorSubcoreMesh(core_axis_name="c", subcore_axis_name="s")

    @functools.partial(
        pl.kernel,
        mesh=mesh,
        compiler_params=pltpu.CompilerParams(use_tc_tiling_on_sc=False),
        out_type=jax.ShapeDtypeStruct((_BATCH, _N_FEATURES, _EMBED), jnp.float32),
        scratch_types=[
            pltpu.VMEM((_PER_W,), jnp.int32),
            pltpu.VMEM((_CHUNK_B * _N_FEATURES, _EMBED), jnp.float32),
            pltpu.SemaphoreType.DMA,
            pltpu.SemaphoreType.DMA,
        ],
    )
    def run(x_hbm, table_hbm, out_hbm, idx_v, rows_v, sem, wsem):
        wid = lax.axis_index("s") * 2 + lax.axis_index("c")
        wstart = wid * _PER_W
        bstart = wid * _B_PER_W

        pltpu.sync_copy(x_hbm.at[pl.ds(wstart, _PER_W)], idx_v)

        lane = lax.iota(jnp.int32, _LANES)

        def add_off(j, _):
            s = j * _LANES
            pos = lane + (wstart + s)
            off = lax.rem(pos, _N_FEATURES) * _N_VALUES
            idx_v[pl.ds(s, _LANES)] = idx_v[pl.ds(s, _LANES)] + off
            return _

        lax.fori_loop(0, _PER_W // _LANES, add_off, None)

        def do_chunk(c, _):
            base = c * (_CHUNK_B * _N_FEATURES)
            copies = [
                pltpu.async_copy(
                    table_hbm.at[idx_v.at[pl.ds(base + g * _G, _G)]],
                    rows_v.at[pl.ds(g * _G, _G)],
                    sem,
                )
                for g in range(_G_PER_CHUNK)
            ]
            for cp in copies:
                cp.wait()

            def fire_row(r, _):
                pltpu.async_copy(
                    rows_v.at[pl.ds(r * _N_FEATURES, _N_FEATURES)],
                    out_hbm.at[bstart + c * _CHUNK_B + r],
                    wsem,
                )
                return _

            lax.fori_loop(0, _CHUNK_B, fire_row, None)

            def drain_row(r, _):
                pltpu.make_async_copy(
                    rows_v.at[pl.ds(0, _N_FEATURES)],
                    out_hbm.at[bstart],
                    wsem,
                ).wait()
                return _

            lax.fori_loop(0, _CHUNK_B, drain_row, None)
            return _

        lax.fori_loop(0, _NCHUNK, do_chunk, None)

    return run


def kernel(x, table):
    xf = x.reshape(-1)
    return _build()(xf, table)


# speedup vs baseline: 1.4530x; 1.1214x over previous
import functools

import jax
import jax.numpy as jnp
from jax import lax
from jax.experimental import pallas as pl
from jax.experimental.pallas import tpu as pltpu
from jax.experimental.pallas import tpu_sc as plsc

_N_FEATURES = 26
_N_VALUES = 38462
_EMBED = 16
_BATCH = 16384

_LANES = 16
_G = 128
_NW = 32
_B_PER_W = _BATCH // _NW
_G_PER_F = _B_PER_W // _G


@functools.cache
def _build():
    mesh = plsc.VectorSubcoreMesh(core_axis_name="c", subcore_axis_name="s")

    @functools.partial(
        pl.kernel,
        mesh=mesh,
        compiler_params=pltpu.CompilerParams(
            use_tc_tiling_on_sc=False, needs_layout_passes=False
        ),
        out_type=jax.ShapeDtypeStruct((_N_FEATURES, _EMBED, _BATCH), jnp.float32),
        scratch_types=[
            pltpu.VMEM((_N_FEATURES, _B_PER_W), jnp.int32),
            pltpu.VMEM((_B_PER_W, _EMBED), jnp.float32),
            pltpu.VMEM((_EMBED, _B_PER_W), jnp.float32),
            pltpu.SemaphoreType.DMA,
        ],
    )
    def run(xt_hbm, table_hbm, out_hbm, idx_all, rows_f, rows_t, sem):
        wid = lax.axis_index("s") * 2 + lax.axis_index("c")
        b0 = wid * _B_PER_W

        pltpu.sync_copy(xt_hbm.at[:, pl.ds(b0, _B_PER_W)], idx_all)

        lane = lax.iota(jnp.int32, _LANES)

        def per_f(f, _):
            seg = table_hbm.at[pl.ds(f * _N_VALUES, _N_VALUES)]
            idx_row = idx_all.at[f]
            copies = [
                pltpu.async_copy(
                    seg.at[idx_row.at[pl.ds(g * _G, _G)]],
                    rows_f.at[pl.ds(g * _G, _G)],
                    sem,
                )
                for g in range(_G_PER_F)
            ]
            for cp in copies:
                cp.wait()

            def tr(r, _):
                v = rows_f[r, :]
                col = jnp.full((_LANES,), r, jnp.int32)
                plsc.store_scatter(rows_t, [lane, col], v)
                return _

            lax.fori_loop(0, _B_PER_W, tr, None)

            pltpu.sync_copy(rows_t, out_hbm.at[f, :, pl.ds(b0, _B_PER_W)])
            return _

        lax.fori_loop(0, _N_FEATURES, per_f, None)

    return run


def kernel(x, table):
    xt = jnp.transpose(x)
    out_k = _build()(xt, table)
    return jnp.transpose(out_k, (2, 0, 1))


# speedup vs baseline: 1.5282x; 1.0518x over previous
import functools

import jax
import jax.numpy as jnp
from jax import lax
from jax.experimental import pallas as pl
from jax.experimental.pallas import tpu as pltpu
from jax.experimental.pallas import tpu_sc as plsc

_N_FEATURES = 26
_N_VALUES = 38462
_EMBED = 16
_BATCH = 16384

_LANES = 16
_G = 128
_NW = 32
_B_PER_W = _BATCH // _NW
_G_PER_F = _B_PER_W // _G


@functools.cache
def _build():
    mesh = plsc.VectorSubcoreMesh(core_axis_name="c", subcore_axis_name="s")

    @functools.partial(
        pl.kernel,
        mesh=mesh,
        compiler_params=pltpu.CompilerParams(
            use_tc_tiling_on_sc=False, needs_layout_passes=False
        ),
        out_type=jax.ShapeDtypeStruct((_N_FEATURES, _EMBED, _BATCH), jnp.float32),
        scratch_types=[
            pltpu.VMEM((_N_FEATURES, _B_PER_W), jnp.int32),
            pltpu.VMEM((_B_PER_W, _EMBED), jnp.float32),
            pltpu.VMEM((_B_PER_W, _EMBED), jnp.float32),
            pltpu.VMEM((_EMBED, _B_PER_W), jnp.float32),
            pltpu.SemaphoreType.DMA,
        ],
    )
    def run(xt_hbm, table_hbm, out_hbm, idx_all, rows_a, rows_b, rows_t, sem):
        wid = lax.axis_index("s") * 2 + lax.axis_index("c")
        b0 = wid * _B_PER_W

        pltpu.sync_copy(xt_hbm.at[:, pl.ds(b0, _B_PER_W)], idx_all)

        lane = lax.iota(jnp.int32, _LANES)

        def fire(f, dst):
            seg = table_hbm.at[pl.ds(f * _N_VALUES, _N_VALUES)]
            idx_row = idx_all.at[f]
            for g in range(_G_PER_F):
                pltpu.async_copy(
                    seg.at[idx_row.at[pl.ds(g * _G, _G)]],
                    dst.at[pl.ds(g * _G, _G)],
                    sem,
                )

        def drain(dst):
            for g in range(_G_PER_F):
                pltpu.make_async_copy(
                    table_hbm.at[pl.ds(0, _G)], dst.at[pl.ds(g * _G, _G)], sem
                ).wait()

        def transpose_store(f, src):
            def tr16(k, _):
                rbase = k * _LANES
                colb = jnp.full((_LANES,), rbase, jnp.int32)
                for i in range(_LANES):
                    v = src[rbase + i, :]
                    plsc.store_scatter(rows_t, [lane, colb + i], v)
                return _

            lax.fori_loop(0, _B_PER_W // _LANES, tr16, None)
            pltpu.sync_copy(rows_t, out_hbm.at[f, :, pl.ds(b0, _B_PER_W)])

        fire(0, rows_a)

        def pair(p, _):
            fa = 2 * p
            drain(rows_a)
            fire(fa + 1, rows_b)
            transpose_store(fa, rows_a)
            drain(rows_b)

            @pl.when(p < _N_FEATURES // 2 - 1)
            def _fire_next():
                fire(fa + 2, rows_a)

            transpose_store(fa + 1, rows_b)
            return _

        lax.fori_loop(0, _N_FEATURES // 2, pair, None)

    return run


def kernel(x, table):
    xt = jnp.transpose(x)
    out_k = _build()(xt, table)
    return jnp.transpose(out_k, (2, 0, 1))


# speedup vs baseline: 1.6014x; 1.0479x over previous
import functools

import jax
import jax.numpy as jnp
from jax import lax
from jax.experimental import pallas as pl
from jax.experimental.pallas import tpu as pltpu
from jax.experimental.pallas import tpu_sc as plsc

_N_FEATURES = 26
_N_VALUES = 38462
_EMBED = 16
_BATCH = 16384

_LANES = 16
_G = 128
_NW = 32
_B_PER_W = _BATCH // _NW
_G_PER_F = _B_PER_W // _G


@functools.cache
def _build():
    mesh = plsc.VectorSubcoreMesh(core_axis_name="c", subcore_axis_name="s")

    @functools.partial(
        pl.kernel,
        mesh=mesh,
        compiler_params=pltpu.CompilerParams(
            use_tc_tiling_on_sc=False, needs_layout_passes=False
        ),
        out_type=jax.ShapeDtypeStruct((_N_FEATURES, _EMBED * _BATCH), jnp.float32),
        scratch_types=[
            pltpu.VMEM((_N_FEATURES, _B_PER_W), jnp.int32),
            pltpu.VMEM((_B_PER_W, _EMBED), jnp.float32),
            pltpu.VMEM((_B_PER_W, _EMBED), jnp.float32),
            pltpu.VMEM((_EMBED * _B_PER_W,), jnp.float32),
            pltpu.SemaphoreType.DMA,
        ],
    )
    def run(xt_hbm, table_hbm, out_hbm, idx_all, rows_a, rows_b, rows_t, sem):
        wid = lax.axis_index("s") * 2 + lax.axis_index("c")
        b0 = wid * _B_PER_W

        pltpu.sync_copy(xt_hbm.at[:, pl.ds(b0, _B_PER_W)], idx_all)

        lane = lax.iota(jnp.int32, _LANES)
        lanepat = (
            lax.shift_right_logical(lane, 3) * (_B_PER_W * 8)
            + (lane & 7) * 128
        )

        def fire(f, dst):
            seg = table_hbm.at[pl.ds(f * _N_VALUES, _N_VALUES)]
            idx_row = idx_all.at[f]
            for g in range(_G_PER_F):
                pltpu.async_copy(
                    seg.at[idx_row.at[pl.ds(g * _G, _G)]],
                    dst.at[pl.ds(g * _G, _G)],
                    sem,
                )

        def drain(dst):
            for g in range(_G_PER_F):
                pltpu.make_async_copy(
                    table_hbm.at[pl.ds(0, _G)], dst.at[pl.ds(g * _G, _G)], sem
                ).wait()

        def transpose_store(f, src):
            def tr16(k, _):
                rbase = k * _LANES
                cb = (rbase // 128) * 1024 + rbase % 128
                for i in range(_LANES):
                    v = src[rbase + i, :]
                    plsc.store_scatter(rows_t, [lanepat + (cb + i)], v)
                return _

            lax.fori_loop(0, _B_PER_W // _LANES, tr16, None)
            for et in range(_EMBED // 8):
                pltpu.sync_copy(
                    rows_t.at[pl.ds(et * (_B_PER_W * 8), _B_PER_W * 8)],
                    out_hbm.at[
                        f,
                        pl.ds(et * (_BATCH * 8) + wid * (_B_PER_W * 8), _B_PER_W * 8),
                    ],
                )

        fire(0, rows_a)

        def pair(p, _):
            fa = 2 * p
            drain(rows_a)
            fire(fa + 1, rows_b)
            transpose_store(fa, rows_a)
            drain(rows_b)

            @pl.when(p < _N_FEATURES // 2 - 1)
            def _fire_next():
                fire(fa + 2, rows_a)

            transpose_store(fa + 1, rows_b)
            return _

        lax.fori_loop(0, _N_FEATURES // 2, pair, None)

    return run


def kernel(x, table):
    xt = jnp.transpose(x)
    out_k = _build()(xt, table)
    k5 = out_k.reshape(_N_FEATURES, 2, _BATCH // 128, 8, 128)
    t = jnp.transpose(k5, (2, 4, 0, 1, 3))
    return t.reshape(_BATCH, _N_FEATURES, _EMBED)
